# baseline (speedup 1.0000x reference)
import functools

import jax
import jax.numpy as jnp
from jax.experimental import pallas as pl
from jax.experimental.pallas import tpu as pltpu
from jax.experimental.pallas import tpu_sc as plsc

NE = 8
NK = 2
BLK = 256
FFN_BLK = 512


def _moe_mm_kernel(em_ref, nxt_ref, xs_ref, w1_hbm, w2_hbm, out_ref,
                   w1c_ref, w2c_ref, stg1_ref, stg2_ref, sem1, sem2):
    i = pl.program_id(0)
    cur = em_ref[i]
    prev = em_ref[jnp.maximum(i - 1, 0)]
    changed = (i == 0) | (cur != prev)

    @pl.when(i == 0)
    def _():
        pltpu.make_async_copy(w1_hbm.at[cur], stg1_ref, sem1).start()
        pltpu.make_async_copy(w2_hbm.at[cur], stg2_ref, sem2).start()

    @pl.when(changed)
    def _():
        pltpu.make_async_copy(w1_hbm.at[cur], stg1_ref, sem1).wait()
        pltpu.make_async_copy(w2_hbm.at[cur], stg2_ref, sem2).wait()
        w1c_ref[...] = stg1_ref[...].astype(jnp.bfloat16)
        w2c_ref[...] = stg2_ref[...].astype(jnp.bfloat16)
        nxt = nxt_ref[i]

        @pl.when(nxt != cur)
        def _():
            pltpu.make_async_copy(w1_hbm.at[nxt], stg1_ref, sem1).start()
            pltpu.make_async_copy(w2_hbm.at[nxt], stg2_ref, sem2).start()

    h = jnp.dot(xs_ref[...], w1c_ref[...], preferred_element_type=jnp.float32)
    h = jax.nn.silu(h).astype(jnp.bfloat16)
    y = jnp.dot(h, w2c_ref[...], preferred_element_type=jnp.float32)
    out_ref[...] = y.astype(jnp.bfloat16)


def _grouped_ffn(blk_expert, nxt_expert, xs, W1, W2, *, captot, hidden, ffn,
                 interpret=False):
    nblk = captot // BLK
    grid_spec = pltpu.PrefetchScalarGridSpec(
        num_scalar_prefetch=2,
        grid=(nblk,),
        in_specs=[
            pl.BlockSpec((BLK, hidden), lambda i, em, nx: (i, 0)),
            pl.BlockSpec(memory_space=pl.ANY),
            pl.BlockSpec(memory_space=pl.ANY),
        ],
        out_specs=pl.BlockSpec((BLK, hidden), lambda i, em, nx: (i, 0)),
        scratch_shapes=[
            pltpu.VMEM((hidden, ffn), jnp.bfloat16),
            pltpu.VMEM((ffn, hidden), jnp.bfloat16),
            pltpu.VMEM((hidden, ffn), jnp.float32),
            pltpu.VMEM((ffn, hidden), jnp.float32),
            pltpu.SemaphoreType.DMA,
            pltpu.SemaphoreType.DMA,
        ],
    )
    return pl.pallas_call(
        _moe_mm_kernel,
        grid_spec=grid_spec,
        out_shape=jax.ShapeDtypeStruct((captot, hidden), jnp.bfloat16),
        compiler_params=pltpu.CompilerParams(
            dimension_semantics=("arbitrary",)),
        interpret=interpret,
    )(blk_expert, nxt_expert, xs, W1, W2)


def _dispatch_scatter(xb, p0, p1, captot):
    T, H = xb.shape
    xi = jax.lax.bitcast_convert_type(
        xb.reshape(T, H // 2, 2), jnp.int32)
    Hw = H // 2
    C = Hw // 128
    W = 128
    nflat = T * C
    chunk = jnp.arange(C, dtype=jnp.int32)[None, :]
    p0f = (p0[:, None] * C + chunk).reshape(1, nflat)
    p1f = (p1[:, None] * C + chunk).reshape(1, nflat)
    xf = xi.reshape(nflat, 128)
    mesh = plsc.VectorSubcoreMesh(core_axis_name="core",
                                  subcore_axis_name="subcore")

    @functools.partial(
        pl.kernel,
        out_type=jax.ShapeDtypeStruct((captot * C, 128), jnp.int32),
        mesh=mesh,
        scratch_types=[],
    )
    def k(x_hbm, p0_hbm, p1_hbm, o_hbm):
        def body(x_vmem, p0_vmem, p1_vmem):
            pltpu.sync_copy(x_vmem, o_hbm.at[p0_vmem.at[0]])
            pltpu.sync_copy(x_vmem, o_hbm.at[p1_vmem.at[0]])

        pltpu.emit_pipeline(
            body,
            grid=(nflat // W,),
            in_specs=[pl.BlockSpec((W, 128), index_map=lambda i: (i, 0)),
                      pl.BlockSpec((1, W), index_map=lambda i: (0, i)),
                      pl.BlockSpec((1, W), index_map=lambda i: (0, i))],
            out_specs=[],
            core_axis_name=("core", "subcore"),
            dimension_semantics=(pltpu.PARALLEL,),
        )(x_hbm, p0_hbm, p1_hbm)

    out = k(xf, p0f, p1f).reshape(captot, Hw)
    return jax.lax.bitcast_convert_type(out, jnp.bfloat16).reshape(captot, H)


def kernel(hidden_states, W_gate, W1, W2, *, interpret=False):
    Bs, Ss, H = hidden_states.shape
    T = Bs * Ss
    E = W_gate.shape[1]
    F = W1.shape[2]
    captot = (NK * T // BLK + NE) * BLK
    nblk = captot // BLK

    x = hidden_states.reshape(T, H)

    logits = x @ W_gate
    i0 = jnp.argmax(logits, axis=-1)
    l0 = jnp.max(logits, axis=-1)
    masked = jnp.where(i0[:, None] == jnp.arange(E)[None, :], -jnp.inf, logits)
    i1 = jnp.argmax(masked, axis=-1)
    l1 = jnp.max(masked, axis=-1)
    w0 = 1.0 / (1.0 + jnp.exp(l1 - l0))
    w1 = 1.0 - w0

    e_all = jnp.stack([i0, i1], axis=1).reshape(-1).astype(jnp.int32)
    t_all = jnp.repeat(jnp.arange(T, dtype=jnp.int32), NK)
    oh = (e_all[:, None] == jnp.arange(NE, dtype=jnp.int32)[None, :])
    oh = oh.astype(jnp.int32)
    cum = jnp.cumsum(oh, axis=0)
    rank = jnp.sum(cum * oh, axis=-1) - 1
    counts = cum[-1]
    pad_counts = ((counts + BLK - 1) // BLK) * BLK
    ends = jnp.cumsum(pad_counts)
    offs = ends - pad_counts
    pos = offs[e_all] + rank
    blk_expert = jnp.searchsorted(
        ends, jnp.arange(nblk, dtype=jnp.int32) * BLK, side="right")
    blk_expert = jnp.minimum(blk_expert, NE - 1).astype(jnp.int32)

    nxt_blk = jnp.minimum(jnp.take(ends // BLK, blk_expert), nblk - 1)
    nxt_expert = jnp.take(blk_expert, nxt_blk).astype(jnp.int32)

    p = pos.reshape(T, NK)
    xs = _dispatch_scatter(x.astype(jnp.bfloat16), p[:, 0], p[:, 1], captot)
    ysw = _grouped_ffn(blk_expert, nxt_expert, xs, W1, W2,
                       captot=captot, hidden=H, ffn=F, interpret=interpret)
    out = w0[:, None] * ysw[p[:, 0]].astype(jnp.float32) \
        + w1[:, None] * ysw[p[:, 1]].astype(jnp.float32)
    return out.reshape(Bs, Ss, H)


# speedup vs baseline: 1.5537x; 1.5537x over previous
import functools

import jax
import jax.numpy as jnp
from jax.experimental import pallas as pl
from jax.experimental.pallas import tpu as pltpu
from jax.experimental.pallas import tpu_sc as plsc

NE = 8
NK = 2
BLK = 256
FFN_BLK = 512


def _moe_mm_kernel(em_ref, nxt_ref, xs_ref, w1_hbm, w2_hbm, out_ref,
                   w1c_ref, w2c_ref, stg1_ref, stg2_ref, sem1, sem2):
    i = pl.program_id(0)
    cur = em_ref[i]
    prev = em_ref[jnp.maximum(i - 1, 0)]
    changed = (i == 0) | (cur != prev)

    @pl.when(i == 0)
    def _():
        pltpu.make_async_copy(w1_hbm.at[cur], stg1_ref, sem1).start()
        pltpu.make_async_copy(w2_hbm.at[cur], stg2_ref, sem2).start()

    @pl.when(changed)
    def _():
        pltpu.make_async_copy(w1_hbm.at[cur], stg1_ref, sem1).wait()
        pltpu.make_async_copy(w2_hbm.at[cur], stg2_ref, sem2).wait()
        w1c_ref[...] = stg1_ref[...].astype(jnp.bfloat16)
        w2c_ref[...] = stg2_ref[...].astype(jnp.bfloat16)
        nxt = nxt_ref[i]

        @pl.when(nxt != cur)
        def _():
            pltpu.make_async_copy(w1_hbm.at[nxt], stg1_ref, sem1).start()
            pltpu.make_async_copy(w2_hbm.at[nxt], stg2_ref, sem2).start()

    h = jnp.dot(xs_ref[...].astype(jnp.bfloat16), w1c_ref[...],
                preferred_element_type=jnp.float32)
    h = jax.nn.silu(h).astype(jnp.bfloat16)
    y = jnp.dot(h, w2c_ref[...], preferred_element_type=jnp.float32)
    out_ref[...] = y.astype(jnp.bfloat16)


def _grouped_ffn(blk_expert, nxt_expert, xs, W1, W2, *, captot, hidden, ffn,
                 interpret=False):
    nblk = captot // BLK
    grid_spec = pltpu.PrefetchScalarGridSpec(
        num_scalar_prefetch=2,
        grid=(nblk,),
        in_specs=[
            pl.BlockSpec((BLK, hidden), lambda i, em, nx: (i, 0)),
            pl.BlockSpec(memory_space=pl.ANY),
            pl.BlockSpec(memory_space=pl.ANY),
        ],
        out_specs=pl.BlockSpec((BLK, hidden), lambda i, em, nx: (i, 0)),
        scratch_shapes=[
            pltpu.VMEM((hidden, ffn), jnp.bfloat16),
            pltpu.VMEM((ffn, hidden), jnp.bfloat16),
            pltpu.VMEM((hidden, ffn), jnp.float32),
            pltpu.VMEM((ffn, hidden), jnp.float32),
            pltpu.SemaphoreType.DMA,
            pltpu.SemaphoreType.DMA,
        ],
    )
    return pl.pallas_call(
        _moe_mm_kernel,
        grid_spec=grid_spec,
        out_shape=jax.ShapeDtypeStruct((captot, hidden), jnp.bfloat16),
        compiler_params=pltpu.CompilerParams(
            dimension_semantics=("arbitrary",)),
        interpret=interpret,
    )(blk_expert, nxt_expert, xs, W1, W2)


def _dispatch_scatter(xb, p0, p1, captot):
    T, H = xb.shape
    C = H // 128
    W = 128
    nflat = T * C
    chunk = jnp.arange(C, dtype=jnp.int32)[None, :]
    p0f = (p0[:, None] * C + chunk).reshape(1, nflat)
    p1f = (p1[:, None] * C + chunk).reshape(1, nflat)
    xf = xb.reshape(nflat, 128)
    mesh = plsc.VectorSubcoreMesh(core_axis_name="core",
                                  subcore_axis_name="subcore")

    @functools.partial(
        pl.kernel,
        out_type=jax.ShapeDtypeStruct((captot * C, 128), jnp.float32),
        mesh=mesh,
        scratch_types=[],
    )
    def k(x_hbm, p0_hbm, p1_hbm, o_hbm):
        def body(x_vmem, p0_vmem, p1_vmem):
            pltpu.sync_copy(x_vmem, o_hbm.at[p0_vmem.at[0]])
            pltpu.sync_copy(x_vmem, o_hbm.at[p1_vmem.at[0]])

        pltpu.emit_pipeline(
            body,
            grid=(nflat // W,),
            in_specs=[pl.BlockSpec((W, 128), index_map=lambda i: (i, 0)),
                      pl.BlockSpec((1, W), index_map=lambda i: (0, i)),
                      pl.BlockSpec((1, W), index_map=lambda i: (0, i))],
            out_specs=[],
            core_axis_name=("core", "subcore"),
            dimension_semantics=(pltpu.PARALLEL,),
        )(x_hbm, p0_hbm, p1_hbm)

    return k(xf, p0f, p1f).reshape(captot, H)


def kernel(hidden_states, W_gate, W1, W2, *, interpret=False):
    Bs, Ss, H = hidden_states.shape
    T = Bs * Ss
    E = W_gate.shape[1]
    F = W1.shape[2]
    captot = (NK * T // BLK + NE) * BLK
    nblk = captot // BLK

    x = hidden_states.reshape(T, H)

    logits = x @ W_gate
    i0 = jnp.argmax(logits, axis=-1)
    l0 = jnp.max(logits, axis=-1)
    masked = jnp.where(i0[:, None] == jnp.arange(E)[None, :], -jnp.inf, logits)
    i1 = jnp.argmax(masked, axis=-1)
    l1 = jnp.max(masked, axis=-1)
    w0 = 1.0 / (1.0 + jnp.exp(l1 - l0))
    w1 = 1.0 - w0

    e_all = jnp.stack([i0, i1], axis=1).reshape(-1).astype(jnp.int32)
    t_all = jnp.repeat(jnp.arange(T, dtype=jnp.int32), NK)
    oh = (e_all[:, None] == jnp.arange(NE, dtype=jnp.int32)[None, :])
    oh = oh.astype(jnp.int32)
    cum = jnp.cumsum(oh, axis=0)
    rank = jnp.sum(cum * oh, axis=-1) - 1
    counts = cum[-1]
    pad_counts = ((counts + BLK - 1) // BLK) * BLK
    ends = jnp.cumsum(pad_counts)
    offs = ends - pad_counts
    pos = offs[e_all] + rank
    blk_expert = jnp.searchsorted(
        ends, jnp.arange(nblk, dtype=jnp.int32) * BLK, side="right")
    blk_expert = jnp.minimum(blk_expert, NE - 1).astype(jnp.int32)

    nxt_blk = jnp.minimum(jnp.take(ends // BLK, blk_expert), nblk - 1)
    nxt_expert = jnp.take(blk_expert, nxt_blk).astype(jnp.int32)

    p = pos.reshape(T, NK)
    xs = _dispatch_scatter(x, p[:, 0], p[:, 1], captot)
    ysw = _grouped_ffn(blk_expert, nxt_expert, xs, W1, W2,
                       captot=captot, hidden=H, ffn=F, interpret=interpret)
    out = w0[:, None] * ysw[p[:, 0]].astype(jnp.float32) \
        + w1[:, None] * ysw[p[:, 1]].astype(jnp.float32)
    return out.reshape(Bs, Ss, H)
